# Initial kernel scaffold; baseline (speedup 1.0000x reference)
import functools

import jax
import jax.numpy as jnp
from jax import lax
from jax.experimental import pallas as pl
from jax.experimental.pallas import tpu as pltpu
from jax.experimental.pallas import tpu_sc as plsc

F32 = jnp.float32
I32 = jnp.int32

N_NODES = 10000
N_PAD = 10240
N_EDGES = 320000
N_TRI = 320000
H = 128
BLK = 512
E_GRID = N_EDGES // BLK
N_GRID = N_PAD // BLK
CHUNK = 128
DR = 16000
NRANGE = N_EDGES // DR


def _swish(v):
    return v * jax.nn.sigmoid(v)


def _rows(cols):
    return pl.BlockSpec((BLK, cols), lambda ii: (ii, 0))


def _full(shape):
    nd = len(shape)
    return pl.BlockSpec(shape, lambda ii: (0,) * nd)


def _nodeemb_body(z_ref, emb_ref, out_ref):
    zb = z_ref[0, 0]
    oh = (zb[:, None] == lax.broadcasted_iota(I32, (1, 95), 1)).astype(F32)
    out_ref[...] = jnp.dot(oh, emb_ref[...], preferred_element_type=F32)


def _node_embed(z3, emb):
    return pl.pallas_call(
        _nodeemb_body,
        grid=(N_GRID,),
        in_specs=[pl.BlockSpec((1, 1, BLK), lambda ii: (ii, 0, 0)), _full((95, H))],
        out_specs=_rows(H),
        out_shape=jax.ShapeDtypeStruct((N_PAD, H), F32),
    )(z3, emb)


def _embed_body(xi, xj, rbf, Wr, br, W1, W2, W3, be, orbf, ox, ot):
    rb = rbf[...]
    re = _swish(jnp.dot(rb, Wr[...], preferred_element_type=F32) + br[...])
    acc = (jnp.dot(xi[...], W1[...], preferred_element_type=F32)
           + jnp.dot(xj[...], W2[...], preferred_element_type=F32)
           + jnp.dot(re, W3[...], preferred_element_type=F32) + be[...])
    xv = _swish(acc)
    ox[...] = xv
    ot[...] = jnp.dot(rb, orbf[...], preferred_element_type=F32) * xv


def _embed(xi, xj, rbf, Wr, br, W1, W2, W3, be, orbf):
    return pl.pallas_call(
        _embed_body,
        grid=(E_GRID,),
        in_specs=[_rows(H), _rows(H), _rows(6), _full((6, H)), _full((1, H)),
                  _full((H, H)), _full((H, H)), _full((H, H)), _full((1, H)),
                  _full((6, H))],
        out_specs=[_rows(H), _rows(H)],
        out_shape=[jax.ShapeDtypeStruct((N_EDGES, H), F32),
                   jax.ShapeDtypeStruct((N_EDGES, H), F32)],
    )(xi, xj, rbf, Wr, br, W1, W2, W3, be, orbf)


def _pre_body(x, rbf, Wji, bji, Wkj, bkj, lr, oji, okj):
    xb = x[...]
    oji[...] = _swish(jnp.dot(xb, Wji[...], preferred_element_type=F32) + bji[...])
    okj[...] = (_swish(jnp.dot(xb, Wkj[...], preferred_element_type=F32) + bkj[...])
                * jnp.dot(rbf[...], lr[...], preferred_element_type=F32))


def _pre(x, rbf, Wji, bji, Wkj, bkj, lr):
    return pl.pallas_call(
        _pre_body,
        grid=(E_GRID,),
        in_specs=[_rows(H), _rows(6), _full((H, H)), _full((1, H)),
                  _full((H, H)), _full((1, H)), _full((6, H))],
        out_specs=[_rows(H), _rows(H)],
        out_shape=[jax.ShapeDtypeStruct((N_EDGES, H), F32),
                   jax.ShapeDtypeStruct((N_EDGES, H), F32)],
    )(x, rbf, Wji, bji, Wkj, bkj, lr)


def _tri_body(g, sbf, ls, U8, out):
    sb = jnp.dot(sbf[...], ls[...], preferred_element_type=F32)
    gb = g[...]
    acc = jnp.zeros((BLK, H), F32)
    for jj in range(8):
        acc = acc + sb[:, jj:jj + 1] * jnp.dot(gb, U8[jj], preferred_element_type=F32)
    out[...] = acc


def _tri(g, sbf, ls, U8):
    return pl.pallas_call(
        _tri_body,
        grid=(E_GRID,),
        in_specs=[_rows(H), _rows(42), _full((42, 8)), _full((8, H, H))],
        out_specs=_rows(H),
        out_shape=jax.ShapeDtypeStruct((N_TRI, H), F32),
    )(g, sbf, ls, U8)


def _post_body(agg, xji, x, rbf, rb0W, rb0b, rb1W, rb1b, WlI, blI,
               q0W, q0b, q1W, q1b, q2W, q2b, q3W, q3b, orbf, ox, ot):
    h = xji[...] + agg[...]
    h1 = _swish(jnp.dot(h, rb0W[...], preferred_element_type=F32) + rb0b[...])
    h = h + _swish(jnp.dot(h1, rb1W[...], preferred_element_type=F32) + rb1b[...])
    h = _swish(jnp.dot(h, WlI[...], preferred_element_type=F32) + blI[...]) + x[...]
    h1 = _swish(jnp.dot(h, q0W[...], preferred_element_type=F32) + q0b[...])
    h = h + _swish(jnp.dot(h1, q1W[...], preferred_element_type=F32) + q1b[...])
    h1 = _swish(jnp.dot(h, q2W[...], preferred_element_type=F32) + q2b[...])
    h = h + _swish(jnp.dot(h1, q3W[...], preferred_element_type=F32) + q3b[...])
    ox[...] = h
    ot[...] = jnp.dot(rbf[...], orbf[...], preferred_element_type=F32) * h


def _post(agg, xji, x, rbf, weights):
    wspecs = [_full((H, H)), _full((1, H))] * 8
    return pl.pallas_call(
        _post_body,
        grid=(E_GRID,),
        in_specs=[_rows(H), _rows(H), _rows(H), _rows(6)] + wspecs + [_full((6, H))],
        out_specs=[_rows(H), _rows(H)],
        out_shape=[jax.ShapeDtypeStruct((N_EDGES, H), F32),
                   jax.ShapeDtypeStruct((N_EDGES, H), F32)],
    )(agg, xji, x, rbf, *weights)


def _nmlp_body(ns, W0, b0, W1, b1, W2, b2, olp, out):
    n = ns[0] + ns[1]
    n = _swish(jnp.dot(n, W0[...], preferred_element_type=F32) + b0[...])
    n = _swish(jnp.dot(n, W1[...], preferred_element_type=F32) + b1[...])
    n = _swish(jnp.dot(n, W2[...], preferred_element_type=F32) + b2[...])
    out[...] = jnp.dot(n, olp[...], preferred_element_type=F32)


def _nmlp(ns, W0, b0, W1, b1, W2, b2, olp):
    return pl.pallas_call(
        _nmlp_body,
        grid=(N_GRID,),
        in_specs=[pl.BlockSpec((2, BLK, H), lambda ii: (0, ii, 0)),
                  _full((H, H)), _full((1, H)), _full((H, H)), _full((1, H)),
                  _full((H, H)), _full((1, H)), _full((H, H))],
        out_specs=_rows(H),
        out_shape=jax.ShapeDtypeStruct((N_PAD, H), F32),
    )(ns, W0, b0, W1, b1, W2, b2, olp)


_MESH = plsc.VectorSubcoreMesh(core_axis_name="c", subcore_axis_name="s")


def _sc_gather(table, idx2):
    nch = idx2.shape[0]

    @functools.partial(
        pl.kernel,
        out_type=jax.ShapeDtypeStruct((nch * CHUNK, H), F32),
        mesh=_MESH,
        scratch_types=[pltpu.VMEM((CHUNK,), I32),
                       pltpu.VMEM((CHUNK, H), F32),
                       pltpu.SemaphoreType.DMA],
    )
    def k(tbl, idx, out, iv, buf, sem):
        wid = lax.axis_index("s") * 2 + lax.axis_index("c")
        nk = (nch - wid + 31) // 32

        def body(it, carry):
            kk = wid + it * 32
            pltpu.sync_copy(idx.at[kk], iv)
            pltpu.async_copy(tbl.at[iv], buf, sem).wait()
            pltpu.sync_copy(buf, out.at[pl.ds(kk * CHUNK, CHUNK)])
            return carry

        lax.fori_loop(0, nk, body, 0)

    return k(table, idx2)


def _sc_scatter_nodes(vals, i2, zrows):
    @functools.partial(
        pl.kernel,
        out_type=jax.ShapeDtypeStruct((2, N_PAD, H), F32),
        mesh=_MESH,
        scratch_types=[pltpu.VMEM((CHUNK,), I32),
                       pltpu.VMEM((CHUNK, H), F32),
                       pltpu.VMEM_SHARED((N_PAD, H), F32),
                       pltpu.SemaphoreType.DMA],
    )
    def k(vref, idx, zr, out, iv, buf, spm, sem):
        c = lax.axis_index("c")
        sid = lax.axis_index("s")
        pltpu.sync_copy(zr.at[pl.ds(sid * 640, 640)], spm.at[pl.ds(sid * 640, 640)])
        plsc.subcore_barrier()
        nk = (1250 - sid + 15) // 16

        def body(it, carry):
            kk = c * 1250 + sid + it * 16
            pltpu.sync_copy(idx.at[kk], iv)
            pltpu.sync_copy(vref.at[pl.ds(kk * CHUNK, CHUNK)], buf)
            pltpu.sync_copy(buf, spm.at[iv], add=True)
            return carry

        lax.fori_loop(0, nk, body, 0)
        plsc.subcore_barrier()
        pltpu.sync_copy(spm.at[pl.ds(sid * 640, 640)],
                        out.at[c, pl.ds(sid * 640, 640)])

    return k(vals, i2, zrows)


def _sc_scatter_edges(m, p2, s2, blo, bhi, zrows):
    @functools.partial(
        pl.kernel,
        out_type=jax.ShapeDtypeStruct((N_EDGES, H), F32),
        mesh=_MESH,
        scratch_types=[pltpu.VMEM((CHUNK,), I32),
                       pltpu.VMEM((CHUNK,), I32),
                       pltpu.VMEM((CHUNK,), I32),
                       pltpu.VMEM((CHUNK, H), F32),
                       pltpu.VMEM((16,), I32),
                       pltpu.VMEM((16,), I32),
                       pltpu.VMEM_SHARED((DR + 8, H), F32),
                       pltpu.SemaphoreType.DMA],
    )
    def k(mref, pref, sref, blo_r, bhi_r, zr, out,
          iv, sv, ldv, buf, blv, bhv, spm, sem):
        c = lax.axis_index("c")
        sid = lax.axis_index("s")
        pltpu.sync_copy(blo_r.at[c], blv)
        pltpu.sync_copy(bhi_r.at[c], bhv)
        blvec = blv[...]
        bhvec = bhv[...]
        lane = lax.iota(I32, 16)
        for r in range(NRANGE // 2):
            base = (c * (NRANGE // 2) + r) * DR
            t0 = jnp.max(jnp.where(lane == r, blvec, 0))
            t1 = jnp.max(jnp.where(lane == r, bhvec, 0))
            pltpu.sync_copy(zr.at[pl.ds(sid * 1000, 1000)],
                            spm.at[pl.ds(sid * 1000, 1000)])
            plsc.subcore_barrier()
            c0 = t0 // CHUNK
            c1 = (t1 + CHUNK - 1) // CHUNK
            nk = jnp.maximum(0, (c1 - c0 - sid + 15) // 16)

            def body(it, carry):
                kk = c0 + sid + it * 16
                pltpu.sync_copy(pref.at[kk], iv)
                pltpu.sync_copy(sref.at[kk], sv)
                for q in range(CHUNK // 16):
                    svq = sv[pl.ds(q * 16, 16)]
                    ok = (svq >= base) & (svq < base + DR)
                    ldv[pl.ds(q * 16, 16)] = jnp.where(ok, svq - base, DR)
                pltpu.async_copy(mref.at[iv], buf, sem).wait()
                pltpu.sync_copy(buf, spm.at[ldv], add=True)
                return carry

            lax.fori_loop(0, nk, body, 0)
            plsc.subcore_barrier()
            pltpu.sync_copy(spm.at[pl.ds(sid * 1000, 1000)],
                            out.at[pl.ds(base + sid * 1000, 1000)])

    return k(m, p2, s2, blo, bhi, zrows)


def kernel(z, rbf, sbf, i, j, idx_kj, idx_ji, params):
    p = params
    i = i.astype(I32)
    j = j.astype(I32)
    idx_kj = idx_kj.astype(I32)
    idx_ji = idx_ji.astype(I32)

    z3 = jnp.pad(z.astype(I32), (0, N_PAD - N_NODES)).reshape(N_GRID, 1, BLK)
    ij2 = jnp.concatenate([i, j]).reshape((2 * N_EDGES) // CHUNK, CHUNK)
    kj2 = idx_kj.reshape(N_TRI // CHUNK, CHUNK)
    i2 = i.reshape(N_EDGES // CHUNK, CHUNK)
    perm = jnp.argsort(idx_ji).astype(I32)
    s_sorted = jnp.take(idx_ji, perm)
    bounds = jnp.searchsorted(s_sorted, jnp.arange(NRANGE + 1) * DR).astype(I32)
    blo = jnp.pad(bounds[:NRANGE].reshape(2, NRANGE // 2), ((0, 0), (0, 6)))
    bhi = jnp.pad(bounds[1:].reshape(2, NRANGE // 2), ((0, 0), (0, 6)))
    p2 = perm.reshape(N_TRI // CHUNK, CHUNK)
    s2 = s_sorted.reshape(N_TRI // CHUNK, CHUNK)
    zrows = jnp.zeros((DR, H), F32)

    def b1(v):
        return v.reshape(1, H)

    W1 = p['emb_lin_W'][0:H]
    W2 = p['emb_lin_W'][H:2 * H]
    W3 = p['emb_lin_W'][2 * H:3 * H]

    xn = _node_embed(z3, p['emb'])
    xij = _sc_gather(xn, ij2)
    xi = xij[:N_EDGES]
    xj = xij[N_EDGES:]
    x, t = _embed(xi, xj, rbf, p['emb_lin_rbf_W'], b1(p['emb_lin_rbf_b']),
                  W1, W2, W3, b1(p['emb_lin_b']), p['out_lin_rbf'][0])

    def out_block(bb, t_e):
        ns = _sc_scatter_nodes(t_e, i2, zrows)
        olp = jnp.pad(p['out_lin'][bb], ((0, 0), (0, H - p['out_lin'][bb].shape[1])))
        return _nmlp(ns, p['out_lins_W'][bb, 0], b1(p['out_lins_b'][bb, 0]),
                     p['out_lins_W'][bb, 1], b1(p['out_lins_b'][bb, 1]),
                     p['out_lins_W'][bb, 2], b1(p['out_lins_b'][bb, 2]), olp)

    P = out_block(0, t)
    for b in range(6):
        xji, xkj = _pre(x, rbf, p['int_lin_ji_W'][b], b1(p['int_lin_ji_b'][b]),
                        p['int_lin_kj_W'][b], b1(p['int_lin_kj_b'][b]),
                        p['int_lin_rbf'][b])
        g = _sc_gather(xkj, kj2)
        U8 = jnp.transpose(p['int_W'][b], (1, 2, 0))
        m = _tri(g, sbf, p['int_lin_sbf'][b], U8)
        agg = _sc_scatter_edges(m, p2, s2, blo, bhi, zrows)
        weights = [
            p['res_before_W'][b, 0, 0], b1(p['res_before_b'][b, 0, 0]),
            p['res_before_W'][b, 0, 1], b1(p['res_before_b'][b, 0, 1]),
            p['int_lin_W'][b], b1(p['int_lin_b'][b]),
            p['res_after_W'][b, 0, 0], b1(p['res_after_b'][b, 0, 0]),
            p['res_after_W'][b, 0, 1], b1(p['res_after_b'][b, 0, 1]),
            p['res_after_W'][b, 1, 0], b1(p['res_after_b'][b, 1, 0]),
            p['res_after_W'][b, 1, 1], b1(p['res_after_b'][b, 1, 1]),
            p['out_lin_rbf'][b + 1],
        ]
        x, t = _post(agg, xji, x, rbf, weights)
        P = P + out_block(b + 1, t)
    return P[:N_NODES, 0:1]


# speedup vs baseline: 1.1840x; 1.1840x over previous
import functools

import jax
import jax.numpy as jnp
from jax import lax
from jax.experimental import pallas as pl
from jax.experimental.pallas import tpu as pltpu
from jax.experimental.pallas import tpu_sc as plsc

F32 = jnp.float32
I32 = jnp.int32

N_NODES = 10000
N_PAD = 10240
N_EDGES = 320000
N_TRI = 320000
H = 128
BLK = 512
E_GRID = N_EDGES // BLK
N_GRID = N_PAD // BLK
CHUNK = 128
DR = 12800
NWIN = N_EDGES // DR
NROUND = 13


def _swish(v):
    return v * jax.nn.sigmoid(v)


def _rows(cols):
    return pl.BlockSpec((BLK, cols), lambda ii: (ii, 0))


def _full(shape):
    nd = len(shape)
    return pl.BlockSpec(shape, lambda ii: (0,) * nd)


def _nodeemb_body(z_ref, emb_ref, out_ref):
    zb = z_ref[0, 0]
    oh = (zb[:, None] == lax.broadcasted_iota(I32, (1, 95), 1)).astype(F32)
    out_ref[...] = jnp.dot(oh, emb_ref[...], preferred_element_type=F32)


def _node_embed(z3, emb):
    return pl.pallas_call(
        _nodeemb_body,
        grid=(N_GRID,),
        in_specs=[pl.BlockSpec((1, 1, BLK), lambda ii: (ii, 0, 0)), _full((95, H))],
        out_specs=_rows(H),
        out_shape=jax.ShapeDtypeStruct((N_PAD, H), F32),
    )(z3, emb)


def _embed_body(xi, xj, rbf, Wr, br, W1, W2, W3, be, orbf, ox, ot):
    rb = rbf[...]
    re = _swish(jnp.dot(rb, Wr[...], preferred_element_type=F32) + br[...])
    acc = (jnp.dot(xi[...], W1[...], preferred_element_type=F32)
           + jnp.dot(xj[...], W2[...], preferred_element_type=F32)
           + jnp.dot(re, W3[...], preferred_element_type=F32) + be[...])
    xv = _swish(acc)
    ox[...] = xv
    ot[...] = jnp.dot(rb, orbf[...], preferred_element_type=F32) * xv


def _embed(xi, xj, rbf, Wr, br, W1, W2, W3, be, orbf):
    return pl.pallas_call(
        _embed_body,
        grid=(E_GRID,),
        in_specs=[_rows(H), _rows(H), _rows(6), _full((6, H)), _full((1, H)),
                  _full((H, H)), _full((H, H)), _full((H, H)), _full((1, H)),
                  _full((6, H))],
        out_specs=[_rows(H), _rows(H)],
        out_shape=[jax.ShapeDtypeStruct((N_EDGES, H), F32),
                   jax.ShapeDtypeStruct((N_EDGES, H), F32)],
    )(xi, xj, rbf, Wr, br, W1, W2, W3, be, orbf)


def _pre_body(x, rbf, Wji, bji, Wkj, bkj, lr, oji, okj):
    xb = x[...]
    oji[...] = _swish(jnp.dot(xb, Wji[...], preferred_element_type=F32) + bji[...])
    okj[...] = (_swish(jnp.dot(xb, Wkj[...], preferred_element_type=F32) + bkj[...])
                * jnp.dot(rbf[...], lr[...], preferred_element_type=F32))


def _pre(x, rbf, Wji, bji, Wkj, bkj, lr):
    return pl.pallas_call(
        _pre_body,
        grid=(E_GRID,),
        in_specs=[_rows(H), _rows(6), _full((H, H)), _full((1, H)),
                  _full((H, H)), _full((1, H)), _full((6, H))],
        out_specs=[_rows(H), _rows(H)],
        out_shape=[jax.ShapeDtypeStruct((N_EDGES, H), F32),
                   jax.ShapeDtypeStruct((N_EDGES, H), F32)],
    )(x, rbf, Wji, bji, Wkj, bkj, lr)


def _tri_body(g, sbf, ls, U8, out):
    sb = jnp.dot(sbf[...], ls[...], preferred_element_type=F32)
    gb = g[...]
    acc = jnp.zeros((BLK, H), F32)
    for jj in range(8):
        acc = acc + sb[:, jj:jj + 1] * jnp.dot(gb, U8[jj], preferred_element_type=F32)
    out[...] = acc


def _tri(g, sbf, ls, U8):
    return pl.pallas_call(
        _tri_body,
        grid=(E_GRID,),
        in_specs=[_rows(H), _rows(42), _full((42, 8)), _full((8, H, H))],
        out_specs=_rows(H),
        out_shape=jax.ShapeDtypeStruct((N_TRI, H), F32),
    )(g, sbf, ls, U8)


def _post_body(agg, xji, x, rbf, rb0W, rb0b, rb1W, rb1b, WlI, blI,
               q0W, q0b, q1W, q1b, q2W, q2b, q3W, q3b, orbf, ox, ot):
    h = xji[...] + agg[...]
    h1 = _swish(jnp.dot(h, rb0W[...], preferred_element_type=F32) + rb0b[...])
    h = h + _swish(jnp.dot(h1, rb1W[...], preferred_element_type=F32) + rb1b[...])
    h = _swish(jnp.dot(h, WlI[...], preferred_element_type=F32) + blI[...]) + x[...]
    h1 = _swish(jnp.dot(h, q0W[...], preferred_element_type=F32) + q0b[...])
    h = h + _swish(jnp.dot(h1, q1W[...], preferred_element_type=F32) + q1b[...])
    h1 = _swish(jnp.dot(h, q2W[...], preferred_element_type=F32) + q2b[...])
    h = h + _swish(jnp.dot(h1, q3W[...], preferred_element_type=F32) + q3b[...])
    ox[...] = h
    ot[...] = jnp.dot(rbf[...], orbf[...], preferred_element_type=F32) * h


def _post(agg, xji, x, rbf, weights):
    wspecs = [_full((H, H)), _full((1, H))] * 7
    return pl.pallas_call(
        _post_body,
        grid=(E_GRID,),
        in_specs=[_rows(H), _rows(H), _rows(H), _rows(6)] + wspecs + [_full((6, H))],
        out_specs=[_rows(H), _rows(H)],
        out_shape=[jax.ShapeDtypeStruct((N_EDGES, H), F32),
                   jax.ShapeDtypeStruct((N_EDGES, H), F32)],
    )(agg, xji, x, rbf, *weights)


def _nmlp_body(ns, W0, b0, W1, b1, W2, b2, olp, out):
    n = ns[0] + ns[1]
    n = _swish(jnp.dot(n, W0[...], preferred_element_type=F32) + b0[...])
    n = _swish(jnp.dot(n, W1[...], preferred_element_type=F32) + b1[...])
    n = _swish(jnp.dot(n, W2[...], preferred_element_type=F32) + b2[...])
    out[...] = jnp.dot(n, olp[...], preferred_element_type=F32)


def _nmlp(ns, W0, b0, W1, b1, W2, b2, olp):
    return pl.pallas_call(
        _nmlp_body,
        grid=(N_GRID,),
        in_specs=[pl.BlockSpec((2, BLK, H), lambda ii: (0, ii, 0)),
                  _full((H, H)), _full((1, H)), _full((H, H)), _full((1, H)),
                  _full((H, H)), _full((1, H)), _full((H, H))],
        out_specs=_rows(H),
        out_shape=jax.ShapeDtypeStruct((N_PAD, H), F32),
    )(ns, W0, b0, W1, b1, W2, b2, olp)


def _mesh():
    return plsc.VectorSubcoreMesh(core_axis_name="c", subcore_axis_name="s")


def _sc_gather(table, idx2):
    nch = idx2.shape[0]

    @functools.partial(
        pl.kernel,
        out_type=jax.ShapeDtypeStruct((nch * CHUNK, H), F32),
        mesh=_mesh(),
        compiler_params=pltpu.CompilerParams(needs_layout_passes=False),
        scratch_types=[pltpu.VMEM((CHUNK,), I32),
                       pltpu.VMEM((CHUNK, H), F32),
                       pltpu.SemaphoreType.DMA],
    )
    def k(tbl, idx, out, iv, buf, sem):
        wid = lax.axis_index("s") * 2 + lax.axis_index("c")
        nk = (nch - wid + 31) // 32

        def body(it, carry):
            kk = wid + it * 32
            pltpu.sync_copy(idx.at[kk], iv)
            pltpu.async_copy(tbl.at[iv], buf, sem).wait()
            pltpu.sync_copy(buf, out.at[pl.ds(kk * CHUNK, CHUNK)])
            return carry

        lax.fori_loop(0, nk, body, 0)

    return k(table, idx2)


def _sc_scatter_nodes(vals, i2, zrows):
    @functools.partial(
        pl.kernel,
        out_type=jax.ShapeDtypeStruct((2, N_PAD, H), F32),
        mesh=_mesh(),
        compiler_params=pltpu.CompilerParams(needs_layout_passes=False),
        scratch_types=[pltpu.VMEM((CHUNK,), I32),
                       pltpu.VMEM((CHUNK, H), F32),
                       pltpu.VMEM_SHARED((N_PAD, H), F32),
                       pltpu.SemaphoreType.DMA],
    )
    def k(vref, idx, zr, out, iv, buf, spm, sem):
        c = lax.axis_index("c")
        sid = lax.axis_index("s")
        pltpu.sync_copy(zr.at[pl.ds(sid * 640, 640)], spm.at[pl.ds(sid * 640, 640)])
        plsc.subcore_barrier()
        nk = (1250 - sid + 15) // 16

        def body(it, carry):
            kk = c * 1250 + sid + it * 16
            pltpu.sync_copy(idx.at[kk], iv)
            pltpu.sync_copy(vref.at[pl.ds(kk * CHUNK, CHUNK)], buf)
            pltpu.sync_copy(buf, spm.at[iv], add=True)
            return carry

        lax.fori_loop(0, nk, body, 0)
        plsc.subcore_barrier()
        pltpu.sync_copy(spm.at[pl.ds(sid * 640, 640)],
                        out.at[c, pl.ds(sid * 640, 640)])

    return k(vals, i2, zrows)


N_CEXP = N_TRI // CHUNK + NWIN


def _sc_scatter_edges(m, pexp, ldexp, blo, bhi, zrows):
    @functools.partial(
        pl.kernel,
        out_type=jax.ShapeDtypeStruct((N_EDGES, H), F32),
        mesh=_mesh(),
        compiler_params=pltpu.CompilerParams(needs_layout_passes=False),
        scratch_types=[pltpu.VMEM((CHUNK,), I32),
                       pltpu.VMEM((CHUNK,), I32),
                       pltpu.VMEM((CHUNK, H), F32),
                       pltpu.VMEM((16,), I32),
                       pltpu.VMEM((16,), I32),
                       pltpu.VMEM_SHARED((DR + 8, H), F32),
                       pltpu.SemaphoreType.DMA],
    )
    def k(mref, pref, ldref, blo_r, bhi_r, zr, out,
          iv, ldv, buf, blv, bhv, spm, sem):
        c = lax.axis_index("c")
        sid = lax.axis_index("s")
        pltpu.sync_copy(blo_r.at[c], blv)
        pltpu.sync_copy(bhi_r.at[c], bhv)
        blvec = blv[...]
        bhvec = bhv[...]
        lane = lax.iota(I32, 16)
        for r in range(NROUND):
            g = c * NROUND + r
            base = g * DR
            c0 = jnp.clip(jnp.max(jnp.where(lane == r, blvec, 0)), 0, N_CEXP)
            c1 = jnp.clip(jnp.max(jnp.where(lane == r, bhvec, 0)), 0, N_CEXP)

            @pl.when(g < NWIN)
            def _round():
                pltpu.sync_copy(zr.at[pl.ds(sid * 800, 800)],
                                spm.at[pl.ds(sid * 800, 800)])
                plsc.subcore_barrier()
                nk = jnp.maximum(0, (c1 - c0 - sid + 15) // 16)

                def body(it, carry):
                    kk = jnp.clip(c0 + sid + it * 16, 0, N_CEXP - 1)
                    pltpu.sync_copy(pref.at[kk], iv)
                    pltpu.sync_copy(ldref.at[kk], ldv)
                    pltpu.async_copy(mref.at[iv], buf, sem).wait()
                    pltpu.sync_copy(buf, spm.at[ldv], add=True)
                    return carry

                lax.fori_loop(0, nk, body, 0)
                plsc.subcore_barrier()
                pltpu.sync_copy(spm.at[pl.ds(sid * 800, 800)],
                                out.at[pl.ds(base + sid * 800, 800)])

    return k(m, pexp, ldexp, blo, bhi, zrows)


def kernel(z, rbf, sbf, i, j, idx_kj, idx_ji, params):
    p = params
    i = i.astype(I32)
    j = j.astype(I32)
    idx_kj = idx_kj.astype(I32)
    idx_ji = idx_ji.astype(I32)

    z3 = jnp.pad(z.astype(I32), (0, N_PAD - N_NODES)).reshape(N_GRID, 1, BLK)
    ij2 = jnp.concatenate([i, j]).reshape((2 * N_EDGES) // CHUNK, CHUNK)
    kj2 = idx_kj.reshape(N_TRI // CHUNK, CHUNK)
    i2 = i.reshape(N_EDGES // CHUNK, CHUNK)
    perm = jnp.argsort(idx_ji).astype(I32)
    s_sorted = jnp.take(idx_ji, perm)
    bounds = jnp.searchsorted(s_sorted, jnp.arange(NWIN + 1) * DR).astype(I32)
    nwin_sz = bounds[1:] - bounds[:-1]
    npad = ((nwin_sz + CHUNK - 1) // CHUNK) * CHUNK
    start_p = jnp.concatenate([jnp.zeros((1,), I32), jnp.cumsum(npad).astype(I32)])
    w_of = s_sorted // DR
    pos = jnp.arange(N_TRI, dtype=I32) - bounds[w_of] + start_p[w_of]
    texp = N_CEXP * CHUNK
    pexp = jnp.zeros((texp,), I32).at[pos].set(perm).reshape(N_CEXP, CHUNK)
    ldexp = jnp.full((texp,), DR, I32).at[pos].set(s_sorted % DR).reshape(N_CEXP, CHUNK)
    cb = start_p // CHUNK
    blo = jnp.stack([jnp.pad(cb[0:NROUND], (0, 16 - NROUND)),
                     jnp.pad(cb[NROUND:NWIN], (0, 16 - (NWIN - NROUND)))])
    bhi = jnp.stack([jnp.pad(cb[1:NROUND + 1], (0, 16 - NROUND)),
                     jnp.pad(cb[NROUND + 1:NWIN + 1], (0, 16 - (NWIN - NROUND)))])
    zrows = jnp.zeros((DR, H), F32)

    def b1(v):
        return v.reshape(1, H)

    W1 = p['emb_lin_W'][0:H]
    W2 = p['emb_lin_W'][H:2 * H]
    W3 = p['emb_lin_W'][2 * H:3 * H]

    xn = _node_embed(z3, p['emb'])
    xij = _sc_gather(xn, ij2)
    xi = xij[:N_EDGES]
    xj = xij[N_EDGES:]
    x, t = _embed(xi, xj, rbf, p['emb_lin_rbf_W'], b1(p['emb_lin_rbf_b']),
                  W1, W2, W3, b1(p['emb_lin_b']), p['out_lin_rbf'][0])

    def out_block(bb, t_e):
        ns = _sc_scatter_nodes(t_e, i2, zrows)
        olp = jnp.pad(p['out_lin'][bb], ((0, 0), (0, H - p['out_lin'][bb].shape[1])))
        return _nmlp(ns, p['out_lins_W'][bb, 0], b1(p['out_lins_b'][bb, 0]),
                     p['out_lins_W'][bb, 1], b1(p['out_lins_b'][bb, 1]),
                     p['out_lins_W'][bb, 2], b1(p['out_lins_b'][bb, 2]), olp)

    P = out_block(0, t)
    for b in range(6):
        xji, xkj = _pre(x, rbf, p['int_lin_ji_W'][b], b1(p['int_lin_ji_b'][b]),
                        p['int_lin_kj_W'][b], b1(p['int_lin_kj_b'][b]),
                        p['int_lin_rbf'][b])
        g = _sc_gather(xkj, kj2)
        U8 = jnp.transpose(p['int_W'][b], (1, 2, 0))
        m = _tri(g, sbf, p['int_lin_sbf'][b], U8)
        agg = _sc_scatter_edges(m, pexp, ldexp, blo, bhi, zrows)
        weights = [
            p['res_before_W'][b, 0, 0], b1(p['res_before_b'][b, 0, 0]),
            p['res_before_W'][b, 0, 1], b1(p['res_before_b'][b, 0, 1]),
            p['int_lin_W'][b], b1(p['int_lin_b'][b]),
            p['res_after_W'][b, 0, 0], b1(p['res_after_b'][b, 0, 0]),
            p['res_after_W'][b, 0, 1], b1(p['res_after_b'][b, 0, 1]),
            p['res_after_W'][b, 1, 0], b1(p['res_after_b'][b, 1, 0]),
            p['res_after_W'][b, 1, 1], b1(p['res_after_b'][b, 1, 1]),
            p['out_lin_rbf'][b + 1],
        ]
        x, t = _post(agg, xji, x, rbf, weights)
        P = P + out_block(b + 1, t)
    return P[:N_NODES, 0:1]
